# baseline (speedup 1.0000x reference)
import functools

import jax
import jax.numpy as jnp
from jax import lax
from jax.experimental import pallas as pl
from jax.experimental.pallas import tpu as pltpu
from jax.experimental.pallas import tpu_sc as plsc

N_UP = 524288
N_DOWN = 262144
D = 128
_C = 128
_L = 16


def _make_kernel():
    info = plsc.get_sparse_core_info()
    nc, ns = info.num_cores, info.num_subcores
    nw = nc * ns
    rows_per_w = N_UP // nw
    n_chunks = rows_per_w // _C
    mesh = plsc.VectorSubcoreMesh(core_axis_name="c", subcore_axis_name="s")

    @functools.partial(
        pl.kernel,
        mesh=mesh,
        out_type=jax.ShapeDtypeStruct((N_UP, D), jnp.float32),
        scratch_types=[
            pltpu.VMEM((_C,), jnp.int32),
            pltpu.VMEM((_C, D), jnp.float32),
            pltpu.VMEM((_C, D), jnp.float32),
            pltpu.SemaphoreType.DMA,
        ],
    )
    def k(res_hbm, down_hbm, idx_hbm, out_hbm, idx_v, gat_v, res_v, sem):
        wid = lax.axis_index("s") * nc + lax.axis_index("c")
        base = wid * rows_per_w

        def body(g, carry):
            r0 = base + g * _C
            pltpu.sync_copy(idx_hbm.at[pl.ds(r0, _C)], idx_v)
            gather = pltpu.async_copy(down_hbm.at[idx_v], gat_v, sem)
            pltpu.sync_copy(res_hbm.at[pl.ds(r0, _C)], res_v)
            gather.wait()

            def add_body(i, c):
                r = i // (D // _L)
                col = (i % (D // _L)) * _L
                gat_v[r, pl.ds(col, _L)] = (
                    gat_v[r, pl.ds(col, _L)] + res_v[r, pl.ds(col, _L)]
                )
                return c

            lax.fori_loop(0, _C * D // _L, add_body, 0)
            pltpu.sync_copy(gat_v, out_hbm.at[pl.ds(r0, _C)])
            return carry

        lax.fori_loop(0, n_chunks, body, 0)

    return k


_unpool = _make_kernel()


def kernel(residual_feat, down_feat, unpool_ind):
    return _unpool(residual_feat, down_feat, unpool_ind.astype(jnp.int32))


# speedup vs baseline: 3.3420x; 3.3420x over previous
import functools

import jax
import jax.numpy as jnp
from jax import lax
from jax.experimental import pallas as pl
from jax.experimental.pallas import tpu as pltpu
from jax.experimental.pallas import tpu_sc as plsc

N_UP = 524288
N_DOWN = 262144
D = 128
_C = 128
_L = 16
_NB = 2


def _make_kernel():
    info = plsc.get_sparse_core_info()
    nc, ns = info.num_cores, info.num_subcores
    nw = nc * ns
    rows_per_w = N_UP // nw
    n_chunks = rows_per_w // _C
    mesh = plsc.VectorSubcoreMesh(core_axis_name="c", subcore_axis_name="s")

    @functools.partial(
        pl.kernel,
        mesh=mesh,
        out_type=jax.ShapeDtypeStruct((N_UP, D), jnp.float32),
        scratch_types=[
            pltpu.VMEM((_NB, _C), jnp.int32),
            pltpu.VMEM((_NB, _C, D), jnp.float32),
            pltpu.VMEM((_NB, _C, D), jnp.float32),
            pltpu.VMEM((_NB, _C, D), jnp.float32),
        ]
        + [pltpu.SemaphoreType.DMA] * (3 * _NB),
    )
    def k(res_hbm, down_hbm, idx_hbm, out_hbm, idx_v, gat_v, res_v, out_v,
          *sems):
        sg = sems[0:_NB]
        sr = sems[_NB:2 * _NB]
        so = sems[2 * _NB:3 * _NB]
        wid = lax.axis_index("s") * nc + lax.axis_index("c")
        base = wid * rows_per_w

        def issue(g, b):
            r0 = base + g * _C
            pltpu.sync_copy(idx_hbm.at[pl.ds(r0, _C)], idx_v.at[b])
            pltpu.async_copy(down_hbm.at[idx_v.at[b]], gat_v.at[b], sg[b])
            pltpu.async_copy(res_hbm.at[pl.ds(r0, _C)], res_v.at[b], sr[b])

        def wait_inputs(b):
            pltpu.make_async_copy(
                down_hbm.at[idx_v.at[b]], gat_v.at[b], sg[b]).wait()
            pltpu.make_async_copy(
                res_hbm.at[pl.ds(base, _C)], res_v.at[b], sr[b]).wait()

        def wait_store(b):
            pltpu.make_async_copy(
                out_v.at[b], out_hbm.at[pl.ds(base, _C)], so[b]).wait()

        def add_chunk(b):
            def row_body(r, c):
                for j in range(D // _L):
                    s = pl.ds(j * _L, _L)
                    out_v[b, r, s] = gat_v[b, r, s] + res_v[b, r, s]
                return c

            lax.fori_loop(0, _C, row_body, 0)

        def store(g, b):
            r0 = base + g * _C
            pltpu.async_copy(out_v.at[b], out_hbm.at[pl.ds(r0, _C)], so[b])

        for b in range(_NB):
            issue(b, b)

        for b in range(_NB):
            wait_inputs(b)
            add_chunk(b)
            store(b, b)
            issue(b + _NB, b)

        def body(gg, c):
            for b in range(_NB):
                g = gg * _NB + b
                wait_inputs(b)
                wait_store(b)
                add_chunk(b)
                store(g, b)

                @pl.when(g + _NB < n_chunks)
                def _():
                    issue(g + _NB, b)

            return c

        lax.fori_loop(1, n_chunks // _NB, body, 0)

        for b in range(_NB):
            wait_store(b)

    return k


_unpool = _make_kernel()


def kernel(residual_feat, down_feat, unpool_ind):
    return _unpool(residual_feat, down_feat, unpool_ind.astype(jnp.int32))


# speedup vs baseline: 3.4226x; 1.0241x over previous
import functools

import jax
import jax.numpy as jnp
from jax import lax
from jax.experimental import pallas as pl
from jax.experimental.pallas import tpu as pltpu
from jax.experimental.pallas import tpu_sc as plsc

N_UP = 524288
N_DOWN = 262144
D = 128
_C = 128
_L = 16
_NB = 2


def _make_kernel():
    info = plsc.get_sparse_core_info()
    nc, ns = info.num_cores, info.num_subcores
    nw = nc * ns
    rows_per_w = N_UP // nw
    n_chunks = rows_per_w // _C
    mesh = plsc.VectorSubcoreMesh(core_axis_name="c", subcore_axis_name="s")

    @functools.partial(
        pl.kernel,
        mesh=mesh,
        out_type=jax.ShapeDtypeStruct((N_UP, D), jnp.float32),
        scratch_types=[
            pltpu.VMEM((rows_per_w // _C, _C), jnp.int32),
            pltpu.VMEM((_NB, _C, D), jnp.float32),
            pltpu.VMEM((_NB, _C, D), jnp.float32),
            pltpu.VMEM((_NB, _C, D), jnp.float32),
        ]
        + [pltpu.SemaphoreType.DMA] * (3 * _NB),
    )
    def k(res_hbm, down_hbm, idx_hbm, out_hbm, idx_v, gat_v, res_v, out_v,
          *sems):
        sg = sems[0:_NB]
        sr = sems[_NB:2 * _NB]
        so = sems[2 * _NB:3 * _NB]
        wid = lax.axis_index("s") * nc + lax.axis_index("c")
        base = wid * rows_per_w

        pltpu.sync_copy(
            idx_hbm.at[pl.ds(wid * n_chunks, n_chunks)], idx_v)

        def issue(g, b):
            r0 = base + g * _C
            pltpu.async_copy(down_hbm.at[idx_v.at[g]], gat_v.at[b], sg[b])
            pltpu.async_copy(res_hbm.at[pl.ds(r0, _C)], res_v.at[b], sr[b])

        def wait_inputs(b):
            pltpu.make_async_copy(
                down_hbm.at[idx_v.at[0]], gat_v.at[b], sg[b]).wait()
            pltpu.make_async_copy(
                res_hbm.at[pl.ds(base, _C)], res_v.at[b], sr[b]).wait()

        def wait_store(b):
            pltpu.make_async_copy(
                out_v.at[b], out_hbm.at[pl.ds(base, _C)], so[b]).wait()

        def add_chunk(b):
            def row_body(r, c):
                for j in range(D // _L):
                    s = pl.ds(j * _L, _L)
                    out_v[b, r, s] = gat_v[b, r, s] + res_v[b, r, s]
                return c

            lax.fori_loop(0, _C, row_body, 0)

        def store(g, b):
            r0 = base + g * _C
            pltpu.async_copy(out_v.at[b], out_hbm.at[pl.ds(r0, _C)], so[b])

        for b in range(_NB):
            issue(b, b)

        for b in range(_NB):
            wait_inputs(b)
            add_chunk(b)
            store(b, b)
            issue(b + _NB, b)

        def body(gg, c):
            for b in range(_NB):
                g = gg * _NB + b
                wait_inputs(b)
                wait_store(b)
                add_chunk(b)
                store(g, b)

                @pl.when(g + _NB < n_chunks)
                def _():
                    issue(g + _NB, b)

            return c

        lax.fori_loop(1, n_chunks // _NB, body, 0)

        for b in range(_NB):
            wait_store(b)

    return k


_unpool = _make_kernel()


def kernel(residual_feat, down_feat, unpool_ind):
    idx2d = unpool_ind.astype(jnp.int32).reshape(N_UP // _C, _C)
    return _unpool(residual_feat, down_feat, idx2d)


# speedup vs baseline: 3.4261x; 1.0010x over previous
import functools

import jax
import jax.numpy as jnp
from jax import lax
from jax.experimental import pallas as pl
from jax.experimental.pallas import tpu as pltpu
from jax.experimental.pallas import tpu_sc as plsc

N_UP = 524288
N_DOWN = 262144
D = 128
_C = 128
_L = 16
_NG = 2
_NR = 4


def _make_kernel():
    info = plsc.get_sparse_core_info()
    nc, ns = info.num_cores, info.num_subcores
    nw = nc * ns
    rows_per_w = N_UP // nw
    n_chunks = rows_per_w // _C
    mesh = plsc.VectorSubcoreMesh(core_axis_name="c", subcore_axis_name="s")

    @functools.partial(
        pl.kernel,
        mesh=mesh,
        out_type=jax.ShapeDtypeStruct((N_UP, D), jnp.float32),
        scratch_types=[
            pltpu.VMEM((rows_per_w // _C, _C), jnp.int32),
            pltpu.VMEM((_NG, _C, D), jnp.float32),
            pltpu.VMEM((_NR, _C, D), jnp.float32),
        ]
        + [pltpu.SemaphoreType.DMA] * (_NG + 2 * _NR),
    )
    def k(res_hbm, down_hbm, idx_hbm, out_hbm, idx_v, gat_v, ro_v, *sems):
        sg = sems[0:_NG]
        sr = sems[_NG:_NG + _NR]
        so = sems[_NG + _NR:_NG + 2 * _NR]
        wid = lax.axis_index("s") * nc + lax.axis_index("c")
        base = wid * rows_per_w

        pltpu.sync_copy(
            idx_hbm.at[pl.ds(wid * n_chunks, n_chunks)], idx_v)

        def issue(g, bg, br):
            r0 = base + g * _C
            pltpu.async_copy(down_hbm.at[idx_v.at[g]], gat_v.at[bg], sg[bg])
            pltpu.async_copy(res_hbm.at[pl.ds(r0, _C)], ro_v.at[br], sr[br])

        def wait_inputs(bg, br):
            pltpu.make_async_copy(
                down_hbm.at[idx_v.at[0]], gat_v.at[bg], sg[bg]).wait()
            pltpu.make_async_copy(
                res_hbm.at[pl.ds(base, _C)], ro_v.at[br], sr[br]).wait()

        def wait_store(br):
            pltpu.make_async_copy(
                ro_v.at[br], out_hbm.at[pl.ds(base, _C)], so[br]).wait()

        def add_chunk(bg, br):
            def row_body(r, c):
                for j in range(D // _L):
                    s = pl.ds(j * _L, _L)
                    plsc.addupdate(ro_v.at[br, r, s], gat_v[bg, r, s])
                return c

            lax.fori_loop(0, _C, row_body, 0)

        def store(g, br):
            r0 = base + g * _C
            pltpu.async_copy(ro_v.at[br], out_hbm.at[pl.ds(r0, _C)], so[br])

        issue(0, 0, 0)
        issue(1, 1, 1)

        for g in range(4):
            bg, br = g % _NG, g % _NR
            wait_inputs(bg, br)
            add_chunk(bg, br)
            store(g, br)
            gn = g + _NG
            brn = gn % _NR
            if gn >= _NR:
                wait_store(brn)
            issue(gn, gn % _NG, brn)

        def body(gg, c):
            for b in range(_NR):
                g = gg * _NR + b
                bg, br = b % _NG, b
                wait_inputs(bg, br)
                add_chunk(bg, br)
                store(g, br)

                @pl.when(g + _NG < n_chunks)
                def _():
                    brn = (b + _NG) % _NR
                    wait_store(brn)
                    issue(g + _NG, bg, brn)

            return c

        lax.fori_loop(1, n_chunks // _NR, body, 0)

        for br in range(_NR):
            wait_store(br)

    return k


_unpool = _make_kernel()


def kernel(residual_feat, down_feat, unpool_ind):
    idx2d = unpool_ind.astype(jnp.int32).reshape(N_UP // _C, _C)
    return _unpool(residual_feat, down_feat, idx2d)
